# baseline (speedup 1.0000x reference)
import functools

import jax
import jax.numpy as jnp
from jax import lax
from jax.experimental import pallas as pl
from jax.experimental.pallas import tpu as pltpu
from jax.experimental.pallas import tpu_sc as plsc

_BATCH = 16384
_D = 64
_NW = 32
_BPW = _BATCH // _NW
_CHUNK = 128
_NCHUNK = _BPW // _CHUNK


def _make_gather():
    mesh = plsc.VectorSubcoreMesh(core_axis_name="c", subcore_axis_name="s")

    @functools.partial(
        pl.kernel,
        mesh=mesh,
        out_type=jax.ShapeDtypeStruct((_BATCH, _D), jnp.float32),
        scratch_types=[
            pltpu.VMEM((_NCHUNK, _CHUNK), jnp.int32),
            pltpu.VMEM((_BPW, _D), jnp.float32),
            pltpu.SemaphoreType.DMA,
        ],
        compiler_params=pltpu.CompilerParams(use_tc_tiling_on_sc=False),
    )
    def k(idx_hbm, table_hbm, out_hbm, idx_v, rows_v, sem):
        wid = lax.axis_index("s") * 2 + lax.axis_index("c")
        pltpu.sync_copy(idx_hbm.at[pl.ds(wid * _NCHUNK, _NCHUNK)], idx_v)
        copies = []
        for j in range(_NCHUNK):
            copies.append(
                pltpu.async_copy(
                    table_hbm.at[idx_v.at[j]],
                    rows_v.at[pl.ds(j * _CHUNK, _CHUNK)],
                    sem,
                )
            )
        for c in copies:
            c.wait()
        pltpu.sync_copy(rows_v, out_hbm.at[pl.ds(wid * _BPW, _BPW)])

    return k


_gather = _make_gather()


def kernel(x, table):
    idx = x.astype(jnp.int32).reshape(_BATCH // _CHUNK, _CHUNK)
    return _gather(idx, table)


# speedup vs baseline: 2.3241x; 2.3241x over previous
import functools

import jax
import jax.numpy as jnp
from jax import lax
from jax.experimental import pallas as pl
from jax.experimental.pallas import tpu as pltpu
from jax.experimental.pallas import tpu_sc as plsc

_BATCH = 16384
_D = 64
_VOCAB = 1000000
_NW = 32
_BPW = _BATCH // _NW
_NBUF = 8


def _make_gather():
    mesh = plsc.VectorSubcoreMesh(core_axis_name="c", subcore_axis_name="s")

    @functools.partial(
        pl.kernel,
        mesh=mesh,
        out_type=jax.ShapeDtypeStruct((_D, _BATCH), jnp.float32),
        scratch_types=[
            pltpu.VMEM((_BPW + 16,), jnp.int32),
            pltpu.VMEM((_NBUF, _D, 128), jnp.float32),
            pltpu.VMEM((_D, _BPW), jnp.float32),
            pltpu.SemaphoreType.DMA,
        ],
        compiler_params=pltpu.CompilerParams(needs_layout_passes=False),
    )
    def k(idx_hbm, table_t_hbm, out_hbm, idx_v, slabs_v, cols_v, sem):
        wid = lax.axis_index("s") * 2 + lax.axis_index("c")
        base = wid * _BPW
        pltpu.sync_copy(idx_hbm.at[pl.ds(base, _BPW)], idx_v.at[pl.ds(0, _BPW)])

        rows4 = [lax.iota(jnp.int32, 16) + 16 * q for q in range(4)]

        def fire(v, b):
            vd = pl.multiple_of(v - lax.rem(v, 128), 128)
            pltpu.async_copy(
                table_t_hbm.at[:, pl.ds(vd, 128)], slabs_v.at[b], sem
            )

        def extract(v, i, b):
            col = jnp.full((16,), lax.rem(v, 128), jnp.int32)
            icol = jnp.full((16,), i, jnp.int32)
            bvec = jnp.full((16,), b, jnp.int32)
            for q in range(4):
                vals = plsc.load_gather(slabs_v, [bvec, rows4[q], col])
                plsc.store_scatter(cols_v, [rows4[q], icol], vals)

        def body(g, _):
            i0 = g * _NBUF
            v16 = idx_v[pl.ds(i0, 16)]
            vs = [v16[b] for b in range(_NBUF)]
            for b in range(_NBUF):
                fire(vs[b], b)
            for b in range(_NBUF):
                pltpu.make_async_copy(
                    table_t_hbm.at[:, pl.ds(0, 128)], slabs_v.at[b], sem
                ).wait()
            for b in range(_NBUF):
                extract(vs[b], i0 + b, b)
            return 0

        lax.fori_loop(0, _BPW // _NBUF, body, 0)
        pltpu.sync_copy(cols_v, out_hbm.at[:, pl.ds(base, _BPW)])

    return k


_gather = _make_gather()


def kernel(x, table):
    out_t = _gather(x.astype(jnp.int32), table.T)
    return out_t.T
